# baseline (speedup 1.0000x reference)
import functools

import jax
import jax.numpy as jnp
from jax import lax
from jax.experimental import pallas as pl
from jax.experimental.pallas import tpu as pltpu
from jax.experimental.pallas import tpu_sc as plsc

D_MODEL = 128
CHUNK = 128
N_GRP = CHUNK // 16
ROWS_W = CHUNK * D_MODEL


@functools.lru_cache(maxsize=None)
def _build(n_rows: int):
    info = plsc.get_sparse_core_info()
    nw = info.num_cores * info.num_subcores
    rows_per_w = n_rows // nw
    n_chunks = rows_per_w // CHUNK
    assert rows_per_w * nw == n_rows and n_chunks * CHUNK == rows_per_w

    mesh = plsc.VectorSubcoreMesh(core_axis_name="c", subcore_axis_name="s")

    @functools.partial(
        pl.kernel,
        mesh=mesh,
        out_type=jax.ShapeDtypeStruct((n_rows * D_MODEL,), jnp.float32),
        scratch_types=[
            pltpu.VMEM((n_chunks, CHUNK), jnp.int32),
            pltpu.VMEM((2 * ROWS_W,), jnp.float32),
            pltpu.VMEM((500 * D_MODEL,), jnp.float32),
            pltpu.SemaphoreType.DMA,
            pltpu.SemaphoreType.DMA,
        ],
    )
    def gather_kernel(idx_hbm, table_hbm, out_hbm, idx_v, rows_v, table_v, s0, s1):
        wid = lax.axis_index("s") * info.num_cores + lax.axis_index("c")
        base = wid * rows_per_w * D_MODEL

        pltpu.sync_copy(table_hbm, table_v)
        pltpu.sync_copy(idx_hbm.at[wid], idx_v)

        ssem = (s0, s1)

        def wait_store(b):
            pltpu.make_async_copy(
                rows_v.at[pl.ds(0, ROWS_W)], out_hbm.at[pl.ds(base, ROWS_W)], ssem[b]
            ).wait()

        def do_chunk(j, b):
            def grp(gi, carry):
                idx16 = idx_v[j, pl.ds(16 * gi, 16)] * D_MODEL
                dst0 = b * ROWS_W + 16 * gi * D_MODEL
                for k in range(16):
                    src = idx16[k]
                    dst = dst0 + k * D_MODEL
                    for g in range(N_GRP):
                        rows_v[pl.ds(dst + 16 * g, 16)] = table_v[
                            pl.ds(src + 16 * g, 16)
                        ]
                return carry

            lax.fori_loop(0, CHUNK // 16, grp, 0)
            pltpu.async_copy(
                rows_v.at[pl.ds(b * ROWS_W, ROWS_W)],
                out_hbm.at[pl.ds(base + j * ROWS_W, ROWS_W)],
                ssem[b],
            )

        def body(gi, carry):
            for b in (0, 1):
                j = 2 * gi + b

                @pl.when(j >= 2)
                def _():
                    wait_store(b)

                do_chunk(j, b)
            return carry

        lax.fori_loop(0, n_chunks // 2, body, 0)
        wait_store(0)
        wait_store(1)

    def run(seq_indices, pe):
        idx = seq_indices.reshape(nw, n_chunks, CHUNK)
        table = pe[0].reshape(-1)
        return gather_kernel(idx, table)

    return run


def kernel(seq_indices, pe):
    b, s = seq_indices.shape
    out = _build(b * s)(seq_indices, pe)
    return out.reshape(b, s, D_MODEL)


# speedup vs baseline: 4.1463x; 4.1463x over previous
import functools

import jax
import jax.numpy as jnp
from jax import lax
from jax.experimental import pallas as pl
from jax.experimental.pallas import tpu as pltpu
from jax.experimental.pallas import tpu_sc as plsc

D_MODEL = 128
CHUNK = 128


@functools.lru_cache(maxsize=None)
def _build(n_rows: int):
    info = plsc.get_sparse_core_info()
    nw = info.num_cores * info.num_subcores
    rows_per_w = n_rows // nw
    n_chunks = rows_per_w // CHUNK
    assert rows_per_w * nw == n_rows and n_chunks * CHUNK == rows_per_w

    mesh = plsc.VectorSubcoreMesh(core_axis_name="c", subcore_axis_name="s")

    @functools.partial(
        pl.kernel,
        mesh=mesh,
        out_type=jax.ShapeDtypeStruct((n_rows, D_MODEL), jnp.float32),
        scratch_types=[
            pltpu.VMEM((n_chunks, CHUNK), jnp.int32),
            pltpu.VMEM((5, CHUNK, D_MODEL), jnp.float32),
            pltpu.VMEM_SHARED((500, D_MODEL), jnp.float32),
            pltpu.SemaphoreType.DMA,
            pltpu.SemaphoreType.DMA,
            pltpu.SemaphoreType.DMA,
            pltpu.SemaphoreType.DMA,
            pltpu.SemaphoreType.DMA,
            pltpu.SemaphoreType.DMA,
            pltpu.SemaphoreType.DMA,
            pltpu.SemaphoreType.DMA,
            pltpu.SemaphoreType.DMA,
            pltpu.SemaphoreType.DMA,
        ],
    )
    def gather_kernel(
        idx_hbm, table_hbm, out_hbm, idx_v, rows_v, table_sh,
        g0, g1, g2, g3, g4, s0, s1, s2, s3, s4,
    ):
        sid = lax.axis_index("s")
        wid = sid * info.num_cores + lax.axis_index("c")
        base = wid * rows_per_w

        @pl.when(sid == 0)
        def _():
            pltpu.sync_copy(table_hbm, table_sh)

        pltpu.sync_copy(idx_hbm.at[wid], idx_v)
        plsc.subcore_barrier()

        NBUF = 5
        AHEAD = 3
        gsem = (g0, g1, g2, g3, g4)
        ssem = (s0, s1, s2, s3, s4)

        def start_gather(j, b):
            pltpu.async_copy(table_sh.at[idx_v.at[j]], rows_v.at[b], gsem[b])

        def wait_gather(b):
            pltpu.make_async_copy(
                table_sh.at[idx_v.at[0]], rows_v.at[b], gsem[b]
            ).wait()

        def wait_store(b):
            pltpu.make_async_copy(
                rows_v.at[b], out_hbm.at[pl.ds(base, CHUNK)], ssem[b]
            ).wait()

        for j0 in range(AHEAD):
            start_gather(j0, j0)

        def body(gi, carry):
            for b in range(NBUF):
                j = NBUF * gi + b
                bn = (b + AHEAD) % NBUF
                wait_gather(b)
                pltpu.async_copy(
                    rows_v.at[b], out_hbm.at[pl.ds(base + j * CHUNK, CHUNK)], ssem[b]
                )

                @pl.when(j + AHEAD < n_chunks)
                def _():
                    @pl.when(j >= NBUF - AHEAD)
                    def _():
                        wait_store(bn)

                    start_gather(j + AHEAD, bn)

            return carry

        lax.fori_loop(0, n_chunks // NBUF, body, 0)
        for k in range(NBUF):
            wait_store((n_chunks - NBUF + k) % NBUF)

    def run(seq_indices, pe):
        idx = seq_indices.reshape(nw, n_chunks, CHUNK)
        table = pe[0]
        return gather_kernel(idx, table)

    return run


def kernel(seq_indices, pe):
    b, s = seq_indices.shape
    out = _build(b * s)(seq_indices, pe)
    return out.reshape(b, s, D_MODEL)


# speedup vs baseline: 4.1518x; 1.0013x over previous
import functools

import jax
import jax.numpy as jnp
from jax import lax
from jax.experimental import pallas as pl
from jax.experimental.pallas import tpu as pltpu
from jax.experimental.pallas import tpu_sc as plsc

D_MODEL = 128
CHUNK = 128


@functools.lru_cache(maxsize=None)
def _build(n_rows: int):
    info = plsc.get_sparse_core_info()
    nw = info.num_cores * info.num_subcores
    rows_per_w = n_rows // nw
    n_chunks = rows_per_w // CHUNK
    assert rows_per_w * nw == n_rows and n_chunks * CHUNK == rows_per_w

    mesh = plsc.VectorSubcoreMesh(core_axis_name="c", subcore_axis_name="s")

    @functools.partial(
        pl.kernel,
        mesh=mesh,
        out_type=jax.ShapeDtypeStruct((n_rows, D_MODEL), jnp.float32),
        scratch_types=[
            pltpu.VMEM((n_chunks, CHUNK), jnp.int32),
            pltpu.VMEM((5, CHUNK, D_MODEL), jnp.float32),
            pltpu.VMEM_SHARED((500, D_MODEL), jnp.float32),
            pltpu.SemaphoreType.DMA,
            pltpu.SemaphoreType.DMA,
            pltpu.SemaphoreType.DMA,
            pltpu.SemaphoreType.DMA,
            pltpu.SemaphoreType.DMA,
            pltpu.SemaphoreType.DMA,
            pltpu.SemaphoreType.DMA,
            pltpu.SemaphoreType.DMA,
            pltpu.SemaphoreType.DMA,
            pltpu.SemaphoreType.DMA,
        ],
    )
    def gather_kernel(
        idx_hbm, table_hbm, out_hbm, idx_v, rows_v, table_sh,
        g0, g1, g2, g3, g4, s0, s1, s2, s3, s4,
    ):
        sid = lax.axis_index("s")
        wid = sid * info.num_cores + lax.axis_index("c")
        base = wid * rows_per_w

        @pl.when(sid == 0)
        def _():
            pltpu.sync_copy(table_hbm, table_sh)

        pltpu.sync_copy(idx_hbm.at[wid], idx_v)
        plsc.subcore_barrier()

        NBUF = 5
        AHEAD = 4
        gsem = (g0, g1, g2, g3, g4)
        ssem = (s0, s1, s2, s3, s4)

        def start_gather(j, b):
            pltpu.async_copy(table_sh.at[idx_v.at[j]], rows_v.at[b], gsem[b])

        def wait_gather(b):
            pltpu.make_async_copy(
                table_sh.at[idx_v.at[0]], rows_v.at[b], gsem[b]
            ).wait()

        def wait_store(b):
            pltpu.make_async_copy(
                rows_v.at[b], out_hbm.at[pl.ds(base, CHUNK)], ssem[b]
            ).wait()

        for j0 in range(AHEAD):
            start_gather(j0, j0)

        def body(gi, carry):
            for b in range(NBUF):
                j = NBUF * gi + b
                bn = (b + AHEAD) % NBUF
                wait_gather(b)
                pltpu.async_copy(
                    rows_v.at[b], out_hbm.at[pl.ds(base + j * CHUNK, CHUNK)], ssem[b]
                )

                @pl.when(j + AHEAD < n_chunks)
                def _():
                    @pl.when(j >= NBUF - AHEAD)
                    def _():
                        wait_store(bn)

                    start_gather(j + AHEAD, bn)

            return carry

        lax.fori_loop(0, n_chunks // NBUF, body, 0)
        for k in range(NBUF):
            wait_store((n_chunks - NBUF + k) % NBUF)

    def run(seq_indices, pe):
        idx = seq_indices.reshape(nw, n_chunks, CHUNK)
        table = pe[0]
        return gather_kernel(idx, table)

    return run


def kernel(seq_indices, pe):
    b, s = seq_indices.shape
    out = _build(b * s)(seq_indices, pe)
    return out.reshape(b, s, D_MODEL)
